# Initial kernel scaffold; baseline (speedup 1.0000x reference)
import jax
import jax.numpy as jnp
from jax.experimental import pallas as pl


def kernel(x, edge_index, join_index, batch, W1l, b1, W1r, W2l, b2, W2r, linW, linb):
    raise NotImplementedError("write your pallas kernel here")


# speedup vs baseline: 3.5140x; 3.5140x over previous
import functools

import jax
import jax.numpy as jnp
from jax import lax
from jax.experimental import pallas as pl
from jax.experimental.pallas import tpu as pltpu
from jax.experimental.pallas import tpu_sc as plsc

N = 10000
E = 320000
D = 128
G = 64

NC = 2
NS = 16
NW = NC * NS
K = 80
CH = E // (NW * K)
ROWS_STEP = 624
ROWS_LEN = 640
ROW_MAX0 = N - ROWS_LEN

_mesh = plsc.VectorSubcoreMesh(core_axis_name="c", subcore_axis_name="s")


def _agg_body(table, src1d, dst1d, z128, parts_out, src_v, dst_v, rows_v, acc, sem):
    cid = lax.axis_index("c")
    sid = lax.axis_index("s")
    wid = cid * NS + sid
    row0 = jnp.minimum(sid * ROWS_STEP, ROW_MAX0)

    pltpu.sync_copy(z128.at[pl.ds(row0, ROWS_LEN)], acc.at[pl.ds(row0, ROWS_LEN)])
    plsc.subcore_barrier()

    base = wid * CH

    def step(i, carry):
        off = (base + i) * K
        pltpu.sync_copy(src1d.at[pl.ds(off, K)], src_v)
        pltpu.sync_copy(dst1d.at[pl.ds(off, K)], dst_v)
        pltpu.async_copy(table.at[src_v], rows_v, sem).wait()
        pltpu.sync_copy(rows_v, acc.at[dst_v], add=True)
        return carry

    lax.fori_loop(0, CH, step, 0)
    plsc.subcore_barrier()

    pltpu.sync_copy(acc.at[pl.ds(row0, ROWS_LEN)],
                    parts_out.at[cid, pl.ds(row0, ROWS_LEN)])


_sc_agg = pl.kernel(
    _agg_body,
    out_type=jax.ShapeDtypeStruct((NC, N, D), jnp.float32),
    mesh=_mesh,
    scratch_types=[
        pltpu.VMEM((K,), jnp.int32),
        pltpu.VMEM((K,), jnp.int32),
        pltpu.VMEM((K, D), jnp.float32),
        pltpu.VMEM_SHARED((N, D), jnp.float32),
        pltpu.SemaphoreType.DMA,
    ],
)


def _deg_body(dst1d, deg_out, dst_v, ones_v, zv, degv, dacc):
    cid = lax.axis_index("c")
    sid = lax.axis_index("s")
    wid = cid * NS + sid
    row0 = jnp.minimum(sid * ROWS_STEP, ROW_MAX0)

    zero16 = jnp.zeros((16,), jnp.float32)
    ones16 = jnp.ones((16,), jnp.float32)
    for i in range(ROWS_LEN):
        zv[i, :] = zero16
    for i in range(K):
        ones_v[i, :] = ones16
    pltpu.sync_copy(zv, dacc.at[pl.ds(row0, ROWS_LEN)])
    plsc.subcore_barrier()

    base = wid * CH

    def step(i, carry):
        off = (base + i) * K
        pltpu.sync_copy(dst1d.at[pl.ds(off, K)], dst_v)
        pltpu.sync_copy(ones_v, dacc.at[dst_v], add=True)
        return carry

    lax.fori_loop(0, CH, step, 0)
    plsc.subcore_barrier()

    pltpu.sync_copy(dacc.at[pl.ds(row0, ROWS_LEN)], zv)
    col0 = jnp.zeros((16,), jnp.int32)
    for b in range(ROWS_LEN // 16):
        rows_idx = jnp.arange(b * 16, (b + 1) * 16, dtype=jnp.int32)
        degv[pl.ds(b * 16, 16)] = zero16 + jnp.float32(b)
    pltpu.sync_copy(degv, deg_out.at[pl.ds(cid * N + row0, ROWS_LEN)])


_sc_deg = pl.kernel(
    _deg_body,
    out_type=jax.ShapeDtypeStruct((NC * N,), jnp.float32),
    mesh=_mesh,
    scratch_types=[
        pltpu.VMEM((K,), jnp.int32),
        pltpu.VMEM((K, 16), jnp.float32),
        pltpu.VMEM((ROWS_LEN, 16), jnp.float32),
        pltpu.VMEM((ROWS_LEN,), jnp.float32),
        pltpu.VMEM_SHARED((N, 16), jnp.float32),
    ],
)

_CONTRACT_LAST = (((1,), (1,)), ((), ()))


def _tc_layer1_body(parts_ref, deg_ref, x_ref, wl_ref, b_ref, wr_ref, o_ref):
    aggsum = parts_ref[0] + parts_ref[1]
    deg = deg_ref[0] + deg_ref[1]
    agg = aggsum / jnp.maximum(deg, 1.0)
    out = (lax.dot_general(agg, wl_ref[...], _CONTRACT_LAST,
                           preferred_element_type=jnp.float32)
           + lax.dot_general(x_ref[...], wr_ref[...], _CONTRACT_LAST,
                             preferred_element_type=jnp.float32)
           + b_ref[...])
    nrm = jnp.sqrt(jnp.sum(out * out, axis=1, keepdims=True))
    o_ref[...] = jnp.maximum(out / jnp.maximum(nrm, 1e-12), 0.0)


def _tc_layer2_body(parts_ref, deg_ref, h_ref, wl_ref, b_ref, wr_ref,
                    batch_ref, linw_ref, linb_ref, o_ref):
    aggsum = parts_ref[0] + parts_ref[1]
    deg = deg_ref[0] + deg_ref[1]
    agg = aggsum / jnp.maximum(deg, 1.0)
    out = (lax.dot_general(agg, wl_ref[...], _CONTRACT_LAST,
                           preferred_element_type=jnp.float32)
           + lax.dot_general(h_ref[...], wr_ref[...], _CONTRACT_LAST,
                             preferred_element_type=jnp.float32)
           + b_ref[...])
    nrm = jnp.sqrt(jnp.sum(out * out, axis=1, keepdims=True))
    h2 = out / jnp.maximum(nrm, 1e-12)
    gids = lax.broadcasted_iota(jnp.int32, (G, N), 0)
    onehot = (gids == batch_ref[...]).astype(jnp.float32)
    counts = jnp.sum(onehot, axis=1, keepdims=True)
    pooled = lax.dot_general(onehot, h2, (((1,), (0,)), ((), ())),
                             preferred_element_type=jnp.float32)
    pooled = pooled / jnp.maximum(counts, 1.0)
    res = jnp.sum(pooled * linw_ref[...], axis=1, keepdims=True) + linb_ref[...]
    o_ref[...] = 1.0 / (1.0 + jnp.exp(-res))


def kernel(x, edge_index, join_index, batch, W1l, b1, W1r, W2l, b2, W2r,
           linW, linb):
    del join_index
    src1d = edge_index[0].astype(jnp.int32).reshape(E)
    dst1d = edge_index[1].astype(jnp.int32).reshape(E)
    z128 = jnp.zeros((N, D), jnp.float32)

    parts1 = _sc_agg(x, src1d, dst1d, z128)
    degj = jax.ops.segment_sum(jnp.ones((E,), jnp.float32), dst1d, num_segments=N)
    deg3 = jnp.stack([degj, jnp.zeros_like(degj)])[:, :, None]
    h1 = pl.pallas_call(
        _tc_layer1_body,
        out_shape=jax.ShapeDtypeStruct((N, D), jnp.float32),
    )(parts1, deg3, x, W1l, b1.reshape(1, D), W1r)

    parts2 = _sc_agg(h1, src1d, dst1d, z128)
    out = pl.pallas_call(
        _tc_layer2_body,
        out_shape=jax.ShapeDtypeStruct((G, 1), jnp.float32),
    )(parts2, deg3, h1, W2l, b2.reshape(1, 2 * D), W2r,
      batch.astype(jnp.int32).reshape(1, N), linW, linb.reshape(1, 1))
    return out
